# baseline (speedup 1.0000x reference)
import functools

import jax
import jax.numpy as jnp
from jax import lax
from jax.experimental import pallas as pl
from jax.experimental.pallas import tpu as pltpu
from jax.experimental.pallas import tpu_sc as plsc

B, L, D, V = 32, 4096, 256, 5
N = B * L

_SC_INFO = plsc.get_sparse_core_info()
NC = _SC_INFO.num_cores
NS = _SC_INFO.num_subcores
NW = NC * NS
TOK_PER_W = N // NW
CT = 128
NCW = TOK_PER_W // CT


def _ln_body(tab_ref, gamma_ref, beta_ref, out_ref):
    tab = tab_ref[...]
    mean = jnp.mean(tab, axis=1, keepdims=True)
    var = jnp.mean((tab - mean) ** 2, axis=1, keepdims=True)
    ntab = (tab - mean) * jax.lax.rsqrt(var + 1e-12)
    out_ref[...] = ntab * gamma_ref[...] + beta_ref[...]


def _normed_table(emb_table, ln_gamma, ln_beta):
    return pl.pallas_call(
        _ln_body,
        out_shape=jax.ShapeDtypeStruct((V, D), jnp.float32),
    )(emb_table, ln_gamma.reshape(1, D), ln_beta.reshape(1, D))


@functools.partial(
    pl.kernel,
    out_type=jax.ShapeDtypeStruct((N, D), jnp.float32),
    mesh=plsc.VectorSubcoreMesh(core_axis_name="c", subcore_axis_name="s"),
    scratch_types=[
        pltpu.VMEM((NCW, CT), jnp.int32),
        pltpu.VMEM((CT, D), jnp.float32),
        pltpu.VMEM((CT, D), jnp.float32),
        pltpu.SemaphoreType.DMA,
        pltpu.SemaphoreType.DMA,
        pltpu.SemaphoreType.DMA,
        pltpu.SemaphoreType.DMA,
    ],
)
def _sc_gather(ntab_hbm, ids_hbm, out_hbm, idx_all, rows0, rows1,
               gsem0, gsem1, osem0, osem1):
    wid = lax.axis_index("s") * NC + lax.axis_index("c")
    tok0 = wid * TOK_PER_W

    pltpu.sync_copy(ids_hbm.at[pl.ds(wid * NCW, NCW)], idx_all)

    def g_start(g, rows, gsem):
        pltpu.make_async_copy(ntab_hbm.at[idx_all.at[g]], rows, gsem).start()

    def g_wait(g, rows, gsem):
        pltpu.make_async_copy(ntab_hbm.at[idx_all.at[g]], rows, gsem).wait()

    def o_start(g, rows, osem):
        pltpu.make_async_copy(
            rows, out_hbm.at[pl.ds(tok0 + g * CT, CT)], osem).start()

    def o_wait(g, rows, osem):
        pltpu.make_async_copy(
            rows, out_hbm.at[pl.ds(tok0 + g * CT, CT)], osem).wait()

    npairs = NCW // 2

    def pair(g2, carry):
        g = g2 * 2

        @pl.when(g2 > 0)
        def _():
            o_wait(g - 2, rows0, osem0)

        g_start(g, rows0, gsem0)

        @pl.when(g2 > 0)
        def _():
            o_wait(g - 1, rows1, osem1)

        g_start(g + 1, rows1, gsem1)

        g_wait(g, rows0, gsem0)
        o_start(g, rows0, osem0)
        g_wait(g + 1, rows1, gsem1)
        o_start(g + 1, rows1, osem1)
        return carry

    lax.fori_loop(0, npairs, pair, 0)
    o_wait(NCW - 2, rows0, osem0)
    o_wait(NCW - 1, rows1, osem1)


@functools.partial(jax.jit, static_argnames=())
def kernel(snp_ids, is_padding, emb_table, ln_gamma, ln_beta):
    ntab = _normed_table(emb_table, ln_gamma, ln_beta)
    out = _sc_gather(ntab, snp_ids.reshape(N // CT, CT))
    return out.reshape(B, L, D), is_padding


# speedup vs baseline: 3.1383x; 3.1383x over previous
import functools

import jax
import jax.numpy as jnp
from jax import lax
from jax.experimental import pallas as pl
from jax.experimental.pallas import tpu as pltpu
from jax.experimental.pallas import tpu_sc as plsc

B, L, D, V = 32, 4096, 256, 5
N = B * L

_SC_INFO = plsc.get_sparse_core_info()
NC = _SC_INFO.num_cores
NS = _SC_INFO.num_subcores
NW = NC * NS
TOK_PER_W = N // NW
CT = 128
NCHUNK = TOK_PER_W // CT


def _ln_body(tab_ref, gamma_ref, beta_ref, out_ref):
    tab = tab_ref[...]
    mean = jnp.mean(tab, axis=1, keepdims=True)
    var = jnp.mean((tab - mean) ** 2, axis=1, keepdims=True)
    ntab = (tab - mean) * jax.lax.rsqrt(var + 1e-12)
    out_ref[...] = ntab * gamma_ref[...] + beta_ref[...]


def _normed_table(emb_table, ln_gamma, ln_beta):
    return pl.pallas_call(
        _ln_body,
        out_shape=jax.ShapeDtypeStruct((V, D), jnp.float32),
    )(emb_table, ln_gamma.reshape(1, D), ln_beta.reshape(1, D))


@functools.partial(
    pl.kernel,
    out_type=jax.ShapeDtypeStruct((N * D,), jnp.float32),
    mesh=plsc.VectorSubcoreMesh(core_axis_name="c", subcore_axis_name="s"),
    scratch_types=[
        pltpu.VMEM((V * D,), jnp.float32),
        pltpu.VMEM((TOK_PER_W,), jnp.int32),
        pltpu.VMEM((CT * D,), jnp.float32),
        pltpu.VMEM((CT * D,), jnp.float32),
        pltpu.SemaphoreType.DMA,
        pltpu.SemaphoreType.DMA,
    ],
)
def _sc_expand(ntab_hbm, ids_hbm, out_hbm, ntab_v, idx_all, rows0, rows1,
               osem0, osem1):
    wid = lax.axis_index("s") * NC + lax.axis_index("c")
    base = wid * TOK_PER_W

    pltpu.sync_copy(ntab_hbm, ntab_v)
    pltpu.sync_copy(ids_hbm.at[pl.ds(base, TOK_PER_W)], idx_all)

    def expand(g, rows, osem):
        def grp(h, carry):
            ids16 = idx_all[pl.ds(g * CT + h * 16, 16)]
            for m in range(16):
                row = pl.multiple_of(ids16[m] * D, 8)
                dst = pl.multiple_of((h * 16 + m) * D, 8)
                for k in range(D // 16):
                    rows[pl.ds(dst + k * 16, 16)] = (
                        ntab_v[pl.ds(row + k * 16, 16)])
            return carry

        lax.fori_loop(0, CT // 16, grp, 0)
        pltpu.make_async_copy(
            rows, out_hbm.at[pl.ds((base + g * CT) * D, CT * D)], osem
        ).start()

    def out_wait(g, rows, osem):
        pltpu.make_async_copy(
            rows, out_hbm.at[pl.ds((base + g * CT) * D, CT * D)], osem
        ).wait()

    npairs = NCHUNK // 2

    def pair(g2, carry):
        g = g2 * 2

        @pl.when(g2 > 0)
        def _():
            out_wait(g - 2, rows0, osem0)

        expand(g, rows0, osem0)

        @pl.when(g2 > 0)
        def _():
            out_wait(g - 1, rows1, osem1)

        expand(g + 1, rows1, osem1)
        return carry

    lax.fori_loop(0, npairs, pair, 0)
    out_wait(NCHUNK - 2, rows0, osem0)
    out_wait(NCHUNK - 1, rows1, osem1)


@functools.partial(jax.jit, static_argnames=())
def kernel(snp_ids, is_padding, emb_table, ln_gamma, ln_beta):
    ntab = _normed_table(emb_table, ln_gamma, ln_beta)
    out = _sc_expand(ntab.reshape(V * D), snp_ids.reshape(N))
    return out.reshape(B, L, D), is_padding


# speedup vs baseline: 3.1999x; 1.0196x over previous
import functools

import jax
import jax.numpy as jnp
from jax import lax
from jax.experimental import pallas as pl
from jax.experimental.pallas import tpu as pltpu
from jax.experimental.pallas import tpu_sc as plsc

B, L, D, V = 32, 4096, 256, 5
N = B * L

_SC_INFO = plsc.get_sparse_core_info()
NC = _SC_INFO.num_cores
NS = _SC_INFO.num_subcores
NW = NC * NS
TOK_PER_W = N // NW
CT = 128
NCHUNK = TOK_PER_W // CT


def _ln_body(tab_ref, gamma_ref, beta_ref, out_ref):
    tab = tab_ref[...]
    mean = jnp.mean(tab, axis=1, keepdims=True)
    var = jnp.mean((tab - mean) ** 2, axis=1, keepdims=True)
    ntab = (tab - mean) * jax.lax.rsqrt(var + 1e-12)
    out_ref[...] = ntab * gamma_ref[...] + beta_ref[...]


def _normed_table(emb_table, ln_gamma, ln_beta):
    return pl.pallas_call(
        _ln_body,
        out_shape=jax.ShapeDtypeStruct((V, D), jnp.float32),
    )(emb_table, ln_gamma.reshape(1, D), ln_beta.reshape(1, D))


@functools.partial(
    pl.kernel,
    out_type=jax.ShapeDtypeStruct((N * D,), jnp.float32),
    mesh=plsc.VectorSubcoreMesh(core_axis_name="c", subcore_axis_name="s"),
    scratch_types=[
        pltpu.VMEM((V * D,), jnp.float32),
        pltpu.VMEM((TOK_PER_W,), jnp.int32),
        pltpu.VMEM((CT * D,), jnp.float32),
        pltpu.VMEM((CT * D,), jnp.float32),
        pltpu.SemaphoreType.DMA,
        pltpu.SemaphoreType.DMA,
    ],
)
def _sc_expand(ntab_hbm, ids_hbm, out_hbm, ntab_v, idx_all, rows0, rows1,
               osem0, osem1):
    wid = lax.axis_index("s") * NC + lax.axis_index("c")
    base = wid * TOK_PER_W

    pltpu.sync_copy(ntab_hbm, ntab_v)
    pltpu.sync_copy(ids_hbm.at[pl.ds(base, TOK_PER_W)], idx_all)

    def expand(g, rows, osem):
        base_t = g * CT

        @plsc.parallel_loop(0, CT // 16, 1, unroll=2)
        def grp(h):
            ids16 = idx_all[pl.ds(base_t + h * 16, 16)]
            for m in range(16):
                row = pl.multiple_of(ids16[m] * D, 8)
                dst = pl.multiple_of((h * 16 + m) * D, 8)
                for k in range(D // 16):
                    rows[pl.ds(dst + k * 16, 16)] = (
                        ntab_v[pl.ds(row + k * 16, 16)])

        pltpu.make_async_copy(
            rows, out_hbm.at[pl.ds((base + g * CT) * D, CT * D)], osem
        ).start()

    def out_wait(g, rows, osem):
        pltpu.make_async_copy(
            rows, out_hbm.at[pl.ds((base + g * CT) * D, CT * D)], osem
        ).wait()

    npairs = NCHUNK // 2

    def pair(g2, carry):
        g = g2 * 2

        @pl.when(g2 > 0)
        def _():
            out_wait(g - 2, rows0, osem0)

        expand(g, rows0, osem0)

        @pl.when(g2 > 0)
        def _():
            out_wait(g - 1, rows1, osem1)

        expand(g + 1, rows1, osem1)
        return carry

    lax.fori_loop(0, npairs, pair, 0)
    out_wait(NCHUNK - 2, rows0, osem0)
    out_wait(NCHUNK - 1, rows1, osem1)


@functools.partial(jax.jit, static_argnames=())
def kernel(snp_ids, is_padding, emb_table, ln_gamma, ln_beta):
    ntab = _normed_table(emb_table, ln_gamma, ln_beta)
    out = _sc_expand(ntab.reshape(V * D), snp_ids.reshape(N))
    return out.reshape(B, L, D), is_padding


# speedup vs baseline: 4.4445x; 1.3889x over previous
import functools

import jax
import jax.numpy as jnp
from jax import lax
from jax.experimental import pallas as pl
from jax.experimental.pallas import tpu as pltpu
from jax.experimental.pallas import tpu_sc as plsc

B, L, D, V = 32, 4096, 256, 5
N = B * L

_SC_INFO = plsc.get_sparse_core_info()
NC = _SC_INFO.num_cores
NS = _SC_INFO.num_subcores
NW = NC * NS
TOK_PER_W = N // NW
CT = 128
NCHUNK = TOK_PER_W // CT
REP = 32


def _ln_body(tab_ref, gamma_ref, beta_ref, out_ref):
    tab = tab_ref[...]
    mean = jnp.mean(tab, axis=1, keepdims=True)
    var = jnp.mean((tab - mean) ** 2, axis=1, keepdims=True)
    ntab = (tab - mean) * jax.lax.rsqrt(var + 1e-12)
    out_ref[...] = ntab * gamma_ref[...] + beta_ref[...]


def _normed_table(emb_table, ln_gamma, ln_beta):
    return pl.pallas_call(
        _ln_body,
        out_shape=jax.ShapeDtypeStruct((V, D), jnp.float32),
    )(emb_table, ln_gamma.reshape(1, D), ln_beta.reshape(1, D))


@functools.partial(
    pl.kernel,
    out_type=jax.ShapeDtypeStruct((B, L, D), jnp.float32),
    mesh=plsc.VectorSubcoreMesh(core_axis_name="c", subcore_axis_name="s"),
    scratch_types=[
        pltpu.VMEM((TOK_PER_W,), jnp.int32),
        pltpu.VMEM((CT,), jnp.int32),
        pltpu.VMEM((CT,), jnp.int32),
        pltpu.VMEM((CT, D), jnp.float32),
        pltpu.VMEM((CT, D), jnp.float32),
        pltpu.SemaphoreType.DMA,
        pltpu.SemaphoreType.DMA,
        pltpu.SemaphoreType.DMA,
        pltpu.SemaphoreType.DMA,
    ],
)
def _sc_expand(tabrep_hbm, ids_hbm, out_hbm, idx_all, idxbuf0, idxbuf1,
               rows0, rows1, gsem0, gsem1, osem0, osem1):
    wid = lax.axis_index("s") * NC + lax.axis_index("c")
    base = wid * TOK_PER_W

    pltpu.sync_copy(ids_hbm.at[pl.ds(base, TOK_PER_W)], idx_all)

    lane = lax.iota(jnp.int32, 16)
    offv = lane * V + (wid % 2) * (16 * V)

    def build_idx(g, idxbuf):
        @plsc.parallel_loop(0, CT // 16, 1, unroll=2)
        def grp(h):
            ids16 = idx_all[pl.ds(g * CT + h * 16, 16)]
            idxbuf[pl.ds(h * 16, 16)] = ids16 + offv

    def gather_start(idxbuf, rows, gsem):
        pltpu.make_async_copy(tabrep_hbm.at[idxbuf], rows, gsem).start()

    def gather_wait(idxbuf, rows, gsem):
        pltpu.make_async_copy(tabrep_hbm.at[idxbuf], rows, gsem).wait()

    def out_start(g, rows, osem):
        pltpu.make_async_copy(
            rows, out_hbm.at[wid, pl.ds(g * CT, CT)], osem
        ).start()

    def out_wait(g, rows, osem):
        pltpu.make_async_copy(
            rows, out_hbm.at[wid, pl.ds(g * CT, CT)], osem
        ).wait()

    npairs = NCHUNK // 2

    def pair(g2, carry):
        g = g2 * 2

        @pl.when(g2 > 0)
        def _():
            out_wait(g - 2, rows0, osem0)

        build_idx(g, idxbuf0)
        gather_start(idxbuf0, rows0, gsem0)

        @pl.when(g2 > 0)
        def _():
            out_wait(g - 1, rows1, osem1)

        build_idx(g + 1, idxbuf1)
        gather_start(idxbuf1, rows1, gsem1)

        gather_wait(idxbuf0, rows0, gsem0)
        out_start(g, rows0, osem0)
        gather_wait(idxbuf1, rows1, gsem1)
        out_start(g + 1, rows1, osem1)
        return carry

    lax.fori_loop(0, npairs, pair, 0)
    out_wait(NCHUNK - 2, rows0, osem0)
    out_wait(NCHUNK - 1, rows1, osem1)


@functools.partial(jax.jit, static_argnames=())
def kernel(snp_ids, is_padding, emb_table, ln_gamma, ln_beta):
    ntab = _normed_table(emb_table, ln_gamma, ln_beta)
    tabrep = jnp.tile(ntab, (REP, 1))
    out = _sc_expand(tabrep, snp_ids.reshape(N))
    return out, is_padding


# speedup vs baseline: 4.4667x; 1.0050x over previous
import functools

import jax
import jax.numpy as jnp
from jax import lax
from jax.experimental import pallas as pl
from jax.experimental.pallas import tpu as pltpu
from jax.experimental.pallas import tpu_sc as plsc

B, L, D, V = 32, 4096, 256, 5
N = B * L

_SC_INFO = plsc.get_sparse_core_info()
NC = _SC_INFO.num_cores
NS = _SC_INFO.num_subcores
NW = NC * NS
TOK_PER_W = N // NW
CT = 64
NCHUNK = TOK_PER_W // CT
NBUF = 4
REP = 32


def _ln_body(tab_ref, gamma_ref, beta_ref, out_ref):
    tab = tab_ref[...]
    mean = jnp.mean(tab, axis=1, keepdims=True)
    var = jnp.mean((tab - mean) ** 2, axis=1, keepdims=True)
    ntab = (tab - mean) * jax.lax.rsqrt(var + 1e-12)
    out_ref[...] = ntab * gamma_ref[...] + beta_ref[...]


def _normed_table(emb_table, ln_gamma, ln_beta):
    return pl.pallas_call(
        _ln_body,
        out_shape=jax.ShapeDtypeStruct((V, D), jnp.float32),
    )(emb_table, ln_gamma.reshape(1, D), ln_beta.reshape(1, D))


@functools.partial(
    pl.kernel,
    out_type=jax.ShapeDtypeStruct((B, L, D), jnp.float32),
    mesh=plsc.VectorSubcoreMesh(core_axis_name="c", subcore_axis_name="s"),
    scratch_types=(
        [pltpu.VMEM((TOK_PER_W,), jnp.int32)]
        + [pltpu.VMEM((CT,), jnp.int32) for _ in range(NBUF)]
        + [pltpu.VMEM((CT, D), jnp.float32) for _ in range(NBUF)]
        + [pltpu.SemaphoreType.DMA for _ in range(2 * NBUF)]
    ),
)
def _sc_expand(tabrep_hbm, ids_hbm, out_hbm, idx_all, *bufs):
    idxbufs = bufs[:NBUF]
    rows = bufs[NBUF:2 * NBUF]
    gsems = bufs[2 * NBUF:3 * NBUF]
    osems = bufs[3 * NBUF:4 * NBUF]
    wid = lax.axis_index("s") * NC + lax.axis_index("c")
    base = wid * TOK_PER_W

    pltpu.sync_copy(ids_hbm.at[pl.ds(base, TOK_PER_W)], idx_all)

    lane = lax.iota(jnp.int32, 16)
    offv = lane * V + (wid % 2) * (16 * V)

    def build_idx(g, idxbuf):
        @plsc.parallel_loop(0, CT // 16, 1, unroll=2)
        def grp(h):
            ids16 = idx_all[pl.ds(g * CT + h * 16, 16)]
            idxbuf[pl.ds(h * 16, 16)] = ids16 + offv

    def gather_start(j):
        pltpu.make_async_copy(
            tabrep_hbm.at[idxbufs[j]], rows[j], gsems[j]
        ).start()

    def gather_wait(j):
        pltpu.make_async_copy(
            tabrep_hbm.at[idxbufs[j]], rows[j], gsems[j]
        ).wait()

    def out_start(g, j):
        pltpu.make_async_copy(
            rows[j], out_hbm.at[wid, pl.ds(g * CT, CT)], osems[j]
        ).start()

    def out_wait(g, j):
        pltpu.make_async_copy(
            rows[j], out_hbm.at[wid, pl.ds(g * CT, CT)], osems[j]
        ).wait()

    def group(q, carry):
        g0 = q * NBUF
        for j in range(NBUF):
            @pl.when(q > 0)
            def _(j=j):
                out_wait(g0 - NBUF + j, j)

            build_idx(g0 + j, idxbufs[j])
            gather_start(j)
        for j in range(NBUF):
            gather_wait(j)
            out_start(g0 + j, j)
        return carry

    lax.fori_loop(0, NCHUNK // NBUF, group, 0)
    for j in range(NBUF):
        out_wait(NCHUNK - NBUF + j, j)


@functools.partial(jax.jit, static_argnames=())
def kernel(snp_ids, is_padding, emb_table, ln_gamma, ln_beta):
    ntab = _normed_table(emb_table, ln_gamma, ln_beta)
    tabrep = jnp.tile(ntab, (REP, 1))
    out = _sc_expand(tabrep, snp_ids.reshape(N))
    return out, is_padding


# speedup vs baseline: 7.1216x; 1.5944x over previous
import functools

import jax
import jax.numpy as jnp
from jax import lax
from jax.experimental import pallas as pl
from jax.experimental.pallas import tpu as pltpu
from jax.experimental.pallas import tpu_sc as plsc

B, L, D, V = 32, 4096, 256, 5
N = B * L

_SC_INFO = plsc.get_sparse_core_info()
NC = _SC_INFO.num_cores
NS = _SC_INFO.num_subcores
NW = NC * NS
TOK_PER_W = N // NW
CT = 64
NCHUNK = TOK_PER_W // CT
NBUF = 4
REP = 32


def _ln_body(tab_ref, gamma_ref, beta_ref, out_ref):
    tab = tab_ref[...]
    mean = jnp.mean(tab, axis=1, keepdims=True)
    var = jnp.mean((tab - mean) ** 2, axis=1, keepdims=True)
    ntab = (tab - mean) * jax.lax.rsqrt(var + 1e-12)
    out_ref[...] = ntab * gamma_ref[...] + beta_ref[...]


def _normed_table(emb_table, ln_gamma, ln_beta):
    return pl.pallas_call(
        _ln_body,
        out_shape=jax.ShapeDtypeStruct((V, D), jnp.float32),
    )(emb_table, ln_gamma.reshape(1, D), ln_beta.reshape(1, D))


@functools.partial(
    pl.kernel,
    out_type=jax.ShapeDtypeStruct((B, L, D), jnp.float32),
    mesh=plsc.VectorSubcoreMesh(core_axis_name="c", subcore_axis_name="s"),
    scratch_types=(
        [pltpu.VMEM((TOK_PER_W,), jnp.int32),
         pltpu.VMEM((V * D,), jnp.float32)]
        + [pltpu.VMEM((CT,), jnp.int32) for _ in range(NBUF)]
        + [pltpu.VMEM((CT, D), jnp.float32) for _ in range(NBUF)]
        + [pltpu.SemaphoreType.DMA for _ in range(2 * NBUF)]
    ),
)
def _sc_expand(tabrep_hbm, ntab_hbm, ids_hbm, out_hbm, idx_all, ntab_v,
               *bufs):
    idxbufs = bufs[:NBUF]
    rows = bufs[NBUF:2 * NBUF]
    gsems = bufs[2 * NBUF:3 * NBUF]
    osems = bufs[3 * NBUF:4 * NBUF]
    wid = lax.axis_index("s") * NC + lax.axis_index("c")
    base = wid * TOK_PER_W

    pltpu.sync_copy(ids_hbm.at[pl.ds(base, TOK_PER_W)], idx_all)
    pltpu.sync_copy(ntab_hbm, ntab_v)

    lane = lax.iota(jnp.int32, 16)
    offv = lane * V + (wid % 2) * (16 * V)

    def build_idx(g, idxbuf):
        @plsc.parallel_loop(0, CT // 16, 1, unroll=2)
        def grp(h):
            ids16 = idx_all[pl.ds(g * CT + h * 16, 16)]
            idxbuf[pl.ds(h * 16, 16)] = ids16 + offv

    def gather_start(j):
        pltpu.make_async_copy(
            tabrep_hbm.at[idxbufs[j]], rows[j], gsems[j]
        ).start()

    def gather_wait(j):
        pltpu.make_async_copy(
            tabrep_hbm.at[idxbufs[j]], rows[j], gsems[j]
        ).wait()

    def out_start(g, j):
        pltpu.make_async_copy(
            rows[j], out_hbm.at[wid, pl.ds(g * CT, CT)], osems[j]
        ).start()

    def out_wait(g, j):
        pltpu.make_async_copy(
            rows[j], out_hbm.at[wid, pl.ds(g * CT, CT)], osems[j]
        ).wait()

    def tec_expand(g, rbuf):
        @plsc.parallel_loop(0, CT // 16, 1, unroll=2)
        def grp(h):
            ids16 = idx_all[pl.ds(g * CT + h * 16, 16)]
            for m in range(16):
                row = pl.multiple_of(ids16[m] * D, 8)
                for k in range(D // 16):
                    rbuf[h * 16 + m, pl.ds(k * 16, 16)] = (
                        ntab_v[pl.ds(row + k * 16, 16)])

    def group(q, carry):
        g0 = q * NBUF
        for j in range(2):
            @pl.when(q > 0)
            def _(j=j):
                out_wait(g0 - NBUF + j, j)

            build_idx(g0 + j, idxbufs[j])
            gather_start(j)
        for j in range(2, NBUF):
            @pl.when(q > 0)
            def _(j=j):
                out_wait(g0 - NBUF + j, j)

            tec_expand(g0 + j, rows[j])
            out_start(g0 + j, j)
        for j in range(2):
            gather_wait(j)
            out_start(g0 + j, j)
        return carry

    lax.fori_loop(0, NCHUNK // NBUF, group, 0)
    for j in range(NBUF):
        out_wait(NCHUNK - NBUF + j, j)


@functools.partial(jax.jit, static_argnames=())
def kernel(snp_ids, is_padding, emb_table, ln_gamma, ln_beta):
    ntab = _normed_table(emb_table, ln_gamma, ln_beta)
    tabrep = jnp.tile(ntab, (REP, 1))
    out = _sc_expand(tabrep, ntab.reshape(V * D), snp_ids.reshape(N))
    return out, is_padding
